# baseline (speedup 1.0000x reference)
import functools

import jax
import jax.numpy as jnp
from jax import lax
from jax.experimental import pallas as pl
from jax.experimental.pallas import tpu as pltpu
from jax.experimental.pallas import tpu_sc as plsc

N_ID = 13
N_DENSE = 13
EMB = 32
BATCH = 16384
VOCAB = 2000
NG = 4
PER_G = 4


def _make_sc_gather(batch: int):
    info = plsc.get_sparse_core_info()
    nw = info.num_cores * info.num_subcores
    n_rows = NG * PER_G * batch
    rows_per_w = n_rows // nw
    n_chunks = 8
    chunk = rows_per_w // n_chunks

    mesh = plsc.VectorSubcoreMesh(core_axis_name="c", subcore_axis_name="s")

    @functools.partial(
        pl.kernel,
        mesh=mesh,
        out_type=jax.ShapeDtypeStruct((n_rows, EMB), jnp.float32),
        scratch_types=[
            pltpu.VMEM((2, chunk), jnp.int32),
            pltpu.VMEM((2, chunk, EMB), jnp.float32),
            pltpu.SemaphoreType.DMA,
            pltpu.SemaphoreType.DMA,
        ],
        compiler_params=pltpu.CompilerParams(use_tc_tiling_on_sc=False,
                                             needs_layout_passes=False),
    )
    def gather_k(table_hbm, idx_hbm, out_hbm, idx_v, rows_v, s0, s1):
        wid = lax.axis_index("s") * info.num_cores + lax.axis_index("c")
        base = wid * rows_per_w
        wsems = [s0, s1]

        def chunk_body(c, slot):
            off = base + c * chunk
            pltpu.sync_copy(idx_hbm.at[pl.ds(off, chunk)], idx_v.at[slot])
            pltpu.async_copy(table_hbm.at[idx_v.at[slot]],
                             rows_v.at[slot], wsems[slot]).wait()
            return pltpu.async_copy(rows_v.at[slot],
                                    out_hbm.at[pl.ds(off, chunk)], wsems[slot])

        cps = [None, None]
        for c in range(n_chunks):
            slot = c % 2
            if cps[slot] is not None:
                cps[slot].wait()
            cps[slot] = chunk_body(c, slot)
        cps[0].wait()
        cps[1].wait()

    return gather_k


@functools.lru_cache(maxsize=None)
def _sc_gather_cached(batch: int):
    return _make_sc_gather(batch)


def _mlp_body(e_ref, d_ref, w1e_ref, w1d_ref, b1_ref, w2_ref, b2_ref,
              w3_ref, b3_ref, out_ref):
    x = jnp.dot(d_ref[...], w1d_ref[...], preferred_element_type=jnp.float32)
    e = jnp.concatenate([e_ref[k].astype(jnp.bfloat16) for k in range(NG)],
                        axis=1)
    x += jnp.dot(e, w1e_ref[...], preferred_element_type=jnp.float32)
    x += b1_ref[...]
    h = jnp.maximum(x, 0.0).astype(jnp.bfloat16)
    h = jnp.maximum(
        jnp.dot(h, w2_ref[...], preferred_element_type=jnp.float32)
        + b2_ref[...], 0.0).astype(jnp.bfloat16)
    out_ref[...] = (
        jnp.dot(h, w3_ref[...], preferred_element_type=jnp.float32)
        + b3_ref[...])


def _mlp(e3, d, w1e, w1d, b1, w2, b2, w3, b3, tb: int = 2048):
    batch = d.shape[0]
    grid = (batch // tb,)
    full2 = lambda shape: pl.BlockSpec(shape, lambda i: (0, 0))
    full3 = lambda shape: pl.BlockSpec(shape, lambda i: (0, 0, 0))
    return pl.pallas_call(
        _mlp_body,
        grid=grid,
        in_specs=[
            pl.BlockSpec((NG, tb, PER_G * EMB), lambda i: (0, i, 0)),
            pl.BlockSpec((tb, N_DENSE), lambda i: (i, 0)),
            full2(w1e.shape),
            full2(w1d.shape),
            full2(b1.shape),
            full2(w2.shape),
            full2(b2.shape),
            full2(w3.shape),
            full2(b3.shape),
        ],
        out_specs=pl.BlockSpec((tb, 1), lambda i: (i, 0)),
        out_shape=jax.ShapeDtypeStruct((batch, 1), jnp.float32),
    )(e3, d, w1e, w1d, b1, w2, b2, w3, b3)


def kernel(id_feature, dense_feature, base_embedding, W1, b1, W2, b2, W3, b3):
    bf = jnp.bfloat16
    ids13 = id_feature[:, :N_ID].astype(jnp.int32)
    ids16 = jnp.concatenate([ids13, ids13[:, :NG * PER_G - N_ID]], axis=1)
    d = dense_feature[:, -N_DENSE:].astype(bf)
    ew = N_ID * EMB
    w1a = W1[:ew] + W1[ew:2 * ew]
    w1d = (W1[2 * ew:2 * ew + N_DENSE] + W1[2 * ew + N_DENSE:]).astype(bf)
    w1e = jnp.pad(w1a, ((0, NG * PER_G * EMB - ew), (0, 0))).astype(bf)
    wargs = (w1e, w1d, b1.reshape(1, -1), W2.astype(bf), b2.reshape(1, -1),
             W3.astype(bf), b3.reshape(1, -1))

    hb = BATCH // 4
    gather = _sc_gather_cached(hb)
    outs = []
    rows = []
    for h in range(4):
        ids_h = (ids16[h * hb:(h + 1) * hb]
                 .reshape(hb, NG, PER_G).transpose(1, 0, 2).reshape(-1))
        rows.append(gather(base_embedding, ids_h))
    for h in range(4):
        e3 = rows[h].reshape(NG, hb, PER_G * EMB)
        outs.append(_mlp(e3, d[h * hb:(h + 1) * hb], *wargs))
    return jnp.concatenate(outs, axis=0)


# speedup vs baseline: 1.0661x; 1.0661x over previous
import functools

import jax
import jax.numpy as jnp
from jax import lax
from jax.experimental import pallas as pl
from jax.experimental.pallas import tpu as pltpu
from jax.experimental.pallas import tpu_sc as plsc

N_ID = 13
N_DENSE = 13
EMB = 32
BATCH = 16384
VOCAB = 2000
NG = 4
PER_G = 4


def _make_sc_gather(batch: int):
    info = plsc.get_sparse_core_info()
    nw = info.num_cores * info.num_subcores
    n_rows = NG * PER_G * batch
    rows_per_w = n_rows // nw
    n_chunks = 8
    chunk = rows_per_w // n_chunks

    mesh = plsc.VectorSubcoreMesh(core_axis_name="c", subcore_axis_name="s")

    @functools.partial(
        pl.kernel,
        mesh=mesh,
        out_type=jax.ShapeDtypeStruct((n_rows, EMB), jnp.float32),
        scratch_types=[
            pltpu.VMEM((2, chunk), jnp.int32),
            pltpu.VMEM((2, chunk, EMB), jnp.float32),
            pltpu.SemaphoreType.DMA,
            pltpu.SemaphoreType.DMA,
        ],
        compiler_params=pltpu.CompilerParams(use_tc_tiling_on_sc=False,
                                             needs_layout_passes=False),
    )
    def gather_k(table_hbm, idx_hbm, out_hbm, idx_v, rows_v, s0, s1):
        wid = lax.axis_index("s") * info.num_cores + lax.axis_index("c")
        base = wid * rows_per_w
        wsems = [s0, s1]

        def chunk_body(c, slot):
            off = base + c * chunk
            pltpu.sync_copy(idx_hbm.at[pl.ds(off, chunk)], idx_v.at[slot])
            pltpu.async_copy(table_hbm.at[idx_v.at[slot]],
                             rows_v.at[slot], wsems[slot]).wait()
            return pltpu.async_copy(rows_v.at[slot],
                                    out_hbm.at[pl.ds(off, chunk)], wsems[slot])

        cps = [None, None]
        for c in range(n_chunks):
            slot = c % 2
            if cps[slot] is not None:
                cps[slot].wait()
            cps[slot] = chunk_body(c, slot)
        cps[0].wait()
        cps[1].wait()

    return gather_k


@functools.lru_cache(maxsize=None)
def _sc_gather_cached(batch: int):
    return _make_sc_gather(batch)


def _mlp_body(e_ref, d_ref, w1e_ref, w1d_ref, b1_ref, w2_ref, b2_ref,
              w3_ref, b3_ref, out_ref):
    x = jnp.dot(d_ref[...], w1d_ref[...], preferred_element_type=jnp.float32)
    e = jnp.concatenate([e_ref[k].astype(jnp.bfloat16) for k in range(NG)],
                        axis=1)
    x += jnp.dot(e, w1e_ref[...], preferred_element_type=jnp.float32)
    x += b1_ref[...]
    h = jnp.maximum(x, 0.0).astype(jnp.bfloat16)
    h = jnp.maximum(
        jnp.dot(h, w2_ref[...], preferred_element_type=jnp.float32)
        + b2_ref[...], 0.0).astype(jnp.bfloat16)
    out_ref[...] = (
        jnp.dot(h, w3_ref[...], preferred_element_type=jnp.float32)
        + b3_ref[...])


def _mlp(e3, d, w1e, w1d, b1, w2, b2, w3, b3, tb: int = 2048):
    batch = d.shape[0]
    grid = (batch // tb,)
    full2 = lambda shape: pl.BlockSpec(shape, lambda i: (0, 0))
    full3 = lambda shape: pl.BlockSpec(shape, lambda i: (0, 0, 0))
    return pl.pallas_call(
        _mlp_body,
        grid=grid,
        in_specs=[
            pl.BlockSpec((NG, tb, PER_G * EMB), lambda i: (0, i, 0)),
            pl.BlockSpec((tb, N_DENSE), lambda i: (i, 0)),
            full2(w1e.shape),
            full2(w1d.shape),
            full2(b1.shape),
            full2(w2.shape),
            full2(b2.shape),
            full2(w3.shape),
            full2(b3.shape),
        ],
        out_specs=pl.BlockSpec((tb, 1), lambda i: (i, 0)),
        out_shape=jax.ShapeDtypeStruct((batch, 1), jnp.float32),
    )(e3, d, w1e, w1d, b1, w2, b2, w3, b3)


def kernel(id_feature, dense_feature, base_embedding, W1, b1, W2, b2, W3, b3):
    bf = jnp.bfloat16
    ids13 = id_feature[:, :N_ID].astype(jnp.int32)
    ids16 = jnp.concatenate([ids13, ids13[:, :NG * PER_G - N_ID]], axis=1)
    d = dense_feature[:, -N_DENSE:].astype(bf)
    ew = N_ID * EMB
    w1a = W1[:ew] + W1[ew:2 * ew]
    w1d = (W1[2 * ew:2 * ew + N_DENSE] + W1[2 * ew + N_DENSE:]).astype(bf)
    w1e = jnp.pad(w1a, ((0, NG * PER_G * EMB - ew), (0, 0))).astype(bf)
    wargs = (w1e, w1d, b1.reshape(1, -1), W2.astype(bf), b2.reshape(1, -1),
             W3.astype(bf), b3.reshape(1, -1))

    hb = BATCH // 2
    gather = _sc_gather_cached(hb)
    outs = []
    rows = []
    for h in range(2):
        ids_h = (ids16[h * hb:(h + 1) * hb]
                 .reshape(hb, NG, PER_G).transpose(1, 0, 2).reshape(-1))
        rows.append(gather(base_embedding, ids_h))
    for h in range(2):
        e3 = rows[h].reshape(NG, hb, PER_G * EMB)
        outs.append(_mlp(e3, d[h * hb:(h + 1) * hb], *wargs))
    return jnp.concatenate(outs, axis=0)
